# baseline (speedup 1.0000x reference)
import functools

import jax
import jax.numpy as jnp
from jax import lax
from jax.experimental import pallas as pl
from jax.experimental.pallas import tpu as pltpu
from jax.experimental.pallas import tpu_sc as plsc

B = 4096
L = 200
N_EMBD = 64
NUM_ROWS = B * L
NC = 2
NS = 16
NW = NC * NS
ROWS_PER_W = NUM_ROWS // NW
IDXW = 128
IDX_ROWS_W = ROWS_PER_W // IDXW
CHUNK = 512
STREAMS = CHUNK // IDXW
NG = ROWS_PER_W // CHUNK
LANES = 16


def _sc_embedding_lookup(table, idx2d, maskf):
    mesh = plsc.VectorSubcoreMesh(core_axis_name="c", subcore_axis_name="s")

    @functools.partial(
        pl.kernel,
        mesh=mesh,
        out_type=jax.ShapeDtypeStruct((NUM_ROWS, N_EMBD), jnp.float32),
        compiler_params=pltpu.CompilerParams(use_tc_tiling_on_sc=False),
        scratch_types=[
            pltpu.VMEM((IDX_ROWS_W, IDXW), jnp.int32),
            pltpu.VMEM((ROWS_PER_W,), jnp.float32),
            pltpu.VMEM((CHUNK, N_EMBD), jnp.float32),
            pltpu.VMEM((CHUNK, N_EMBD), jnp.float32),
            pltpu.SemaphoreType.DMA,
            pltpu.SemaphoreType.DMA,
            pltpu.SemaphoreType.DMA,
            pltpu.SemaphoreType.DMA,
        ],
    )
    def k(table_hbm, idx_hbm, mask_hbm, out_hbm,
          idx_v, mask_v, rows0, rows1, sg0, sg1, sw0, sw1):
        wid = lax.axis_index("s") * NC + lax.axis_index("c")
        row0 = wid * ROWS_PER_W
        rows = (rows0, rows1)
        sg = (sg0, sg1)
        sw = (sw0, sw1)

        pltpu.sync_copy(idx_hbm.at[pl.ds(wid * IDX_ROWS_W, IDX_ROWS_W)], idx_v)
        pltpu.sync_copy(mask_hbm.at[pl.ds(row0, ROWS_PER_W)], mask_v)

        def fire_gathers(g, b):
            for j in range(STREAMS):
                pltpu.async_copy(
                    table_hbm.at[idx_v.at[STREAMS * g + j]],
                    rows[b].at[pl.ds(j * IDXW, IDXW)],
                    sg[b],
                )

        def wait_gathers(b):
            pltpu.make_async_copy(
                out_hbm.at[pl.ds(0, CHUNK)], rows[b], sg[b]
            ).wait()

        def fire_writeback(g, b):
            pltpu.async_copy(
                rows[b], out_hbm.at[pl.ds(row0 + g * CHUNK, CHUNK)], sw[b]
            )

        def wait_writeback(b):
            pltpu.make_async_copy(
                rows[b], out_hbm.at[pl.ds(0, CHUNK)], sw[b]
            ).wait()

        def multiply(g, b):
            def grp_body(grp, _):
                mvec = mask_v[pl.ds(g * CHUNK + grp * LANES, LANES)]
                for j in range(LANES):
                    m = mvec[j]
                    r = grp * LANES + j
                    for c in range(N_EMBD // LANES):
                        sl = pl.ds(c * LANES, LANES)
                        rows[b][r, sl] = rows[b][r, sl] * m
                return 0

            lax.fori_loop(0, CHUNK // LANES, grp_body, 0)

        def body(g, b, fire_next, wait_prev_wb):
            wait_gathers(b)
            multiply(g, b)
            fire_writeback(g, b)
            if wait_prev_wb:
                wait_writeback(1 - b)
            if fire_next:
                fire_gathers(g + 1, 1 - b)

        fire_gathers(0, 0)
        wait_gathers(0)
        multiply(0, 0)
        fire_writeback(0, 0)
        fire_gathers(1, 1)

        def pair_body(i, _):
            for db in range(2):
                g = 2 * i + 1 + db
                body(g, (1 + db) % 2, fire_next=True, wait_prev_wb=True)
            return 0

        lax.fori_loop(0, (NG - 4) // 2, pair_body, 0)

        body(NG - 3, 1, fire_next=True, wait_prev_wb=True)
        body(NG - 2, 0, fire_next=True, wait_prev_wb=True)
        body(NG - 1, 1, fire_next=False, wait_prev_wb=True)
        wait_writeback(1)

    return k(table, idx2d, maskf)


def kernel(x, attention_mask, table):
    idx2d = x.reshape(NUM_ROWS // IDXW, IDXW)
    maskf = attention_mask.reshape(NUM_ROWS)
    out = _sc_embedding_lookup(table, idx2d, maskf)
    return out.reshape(B, L, N_EMBD)
